# baseline (speedup 1.0000x reference)
import functools

import jax
import jax.numpy as jnp
from jax import lax
from jax.experimental import pallas as pl
from jax.experimental.pallas import tpu as pltpu
from jax.experimental.pallas import tpu_sc as plsc

MAX_R = 0.1
YNORM = 1.0
P = 10

_SC_MESH = plsc.VectorSubcoreMesh(core_axis_name="c", subcore_axis_name="s")
_SC_PARAMS = pltpu.CompilerParams(needs_layout_passes=False)


def _make_gather(N, off, EC):
    C = 8000
    layout = [(0, 10), (10, 10), (20, 5), (25, 5)]
    for _, cnt in layout:
        assert EC % cnt == 0 and (EC // cnt) % C == 0

    ev = jax.ShapeDtypeStruct((EC,), jnp.float32)

    @functools.partial(
        pl.kernel,
        out_type=(ev, ev, ev, ev),
        mesh=_SC_MESH,
        compiler_params=_SC_PARAMS,
        scratch_types=[
            pltpu.VMEM((N,), jnp.float32),
            pltpu.VMEM((C,), jnp.int32),
            pltpu.VMEM((C,), jnp.int32),
            pltpu.VMEM((C,), jnp.float32),
        ],
    )
    def gather_kernel(px, py, a0h, a1h, src_hbm, dst_hbm,
                      odx, ody, oa0, oa1, tab_v, ids_v, idd_v, out_v):
        wid = lax.axis_index("c") * 16 + lax.axis_index("s")
        tabs = [px, py, a0h, a1h]
        outs = [odx, ody, oa0, oa1]

        for k, (start, cnt) in enumerate(layout):
            @pl.when((wid >= start) & (wid < start + cnt))
            def _(k=k, start=start, cnt=cnt):
                sid = wid - start
                shard = EC // cnt
                nch = shard // C
                pltpu.sync_copy(tabs[k], tab_v)

                @pl.loop(0, nch)
                def _(ci):
                    base = sid * shard + ci * C
                    pltpu.sync_copy(dst_hbm.at[pl.ds(off + base, C)], idd_v)
                    if k < 2:
                        pltpu.sync_copy(src_hbm.at[pl.ds(off + base, C)],
                                        ids_v)

                        @plsc.parallel_loop(0, C, step=16, unroll=8)
                        def _(o):
                            vs = plsc.load_gather(tab_v, [ids_v[pl.ds(o, 16)]])
                            vd = plsc.load_gather(tab_v, [idd_v[pl.ds(o, 16)]])
                            out_v[pl.ds(o, 16)] = (vs - vd) * (1.0 / MAX_R)
                    else:
                        @plsc.parallel_loop(0, C, step=16, unroll=8)
                        def _(o):
                            out_v[pl.ds(o, 16)] = plsc.load_gather(
                                tab_v, [idd_v[pl.ds(o, 16)]])
                    pltpu.sync_copy(out_v, outs[k].at[pl.ds(base, C)])

    return gather_kernel


def _make_mlp(EC):
    BLK = 5120
    Q = BLK // 4
    assert EC % BLK == 0

    def mlp_body(dx_ref, dy_ref, a0_ref, a1_ref,
                 w1_ref, w2_ref, b2_ref, w3_ref, b3_ref,
                 ox_ref, oy_ref):
        d0 = dx_ref[...].reshape(1, BLK)
        d1 = dy_ref[...].reshape(1, BLK)
        r = jnp.sqrt(d0 * d0 + d1 * d1)
        fp = jnp.concatenate(
            [d0, d1, r,
             a0_ref[...].reshape(1, BLK), a1_ref[...].reshape(1, BLK),
             jnp.ones((1, BLK), jnp.float32)], axis=0)
        dn = (((1,), (0,)), ((), ()))
        h = lax.dot_general(w1_ref[...], fp.astype(jnp.bfloat16), dn,
                            preferred_element_type=jnp.float32)
        h = jnp.maximum(h.astype(jnp.bfloat16), 0)
        h = lax.dot_general(w2_ref[...], h, dn,
                            preferred_element_type=jnp.float32)
        h = jnp.maximum(h.astype(jnp.bfloat16) + b2_ref[...], 0)
        o = lax.dot_general(w3_ref[...], h, dn,
                            preferred_element_type=jnp.float32) + b3_ref[...]
        ox_ref[...] = o[0, :]
        oy_ref[...] = o[1, :]

    ev = jax.ShapeDtypeStruct((EC,), jnp.float32)
    edge_spec = pl.BlockSpec((BLK,), lambda i: (i,))
    full = lambda s: pl.BlockSpec(s, lambda i: (0, 0))
    return pl.pallas_call(
        mlp_body,
        grid=(EC // BLK,),
        in_specs=[edge_spec, edge_spec, edge_spec, edge_spec,
                  full((64, 6)), full((64, 64)),
                  full((64, 1)), full((8, 64)), full((8, 1))],
        out_specs=(edge_spec, edge_spec),
        out_shape=(ev, ev),
    )


def _make_scatter(N, off, EC, first, last):
    shard = EC // 16
    C = 8000
    nch = shard // C
    assert shard % C == 0 and C % 16 == 0
    SL = -(-N // 16)
    SL = (SL + 7) // 8 * 8
    NPAD = SL * 16

    parts = jax.ShapeDtypeStruct((32 * NPAD,), jnp.float32)
    out_type = (parts, jax.ShapeDtypeStruct((2 * NPAD,), jnp.float32)) \
        if last else (parts,)

    @functools.partial(
        pl.kernel,
        out_type=out_type,
        mesh=_SC_MESH,
        compiler_params=_SC_PARAMS,
        scratch_types=[
            pltpu.VMEM((NPAD,), jnp.float32),
            pltpu.VMEM((C,), jnp.int32),
            pltpu.VMEM((C,), jnp.float32),
            pltpu.VMEM((SL,), jnp.float32),
            pltpu.VMEM((SL,), jnp.float32),
        ],
    )
    def scatter_kernel(mx_hbm, my_hbm, dst_hbm, init_hbm, *refs):
        if last:
            part_hbm, out_hbm, acc_v, dst_v, msg_v, red_v, tmp_v = refs
        else:
            part_hbm, acc_v, dst_v, msg_v, red_v, tmp_v = refs
        c = lax.axis_index("c")
        sid = lax.axis_index("s")
        wid = c * 16 + sid
        if first:
            pltpu.sync_copy(init_hbm, acc_v)
        else:
            pltpu.sync_copy(init_hbm.at[pl.ds(wid * NPAD, NPAD)], acc_v)

        @pl.loop(0, nch)
        def _(ci):
            base = sid * shard + ci * C
            pltpu.sync_copy(dst_hbm.at[pl.ds(off + base, C)], dst_v)

            @pl.when(c == 0)
            def _():
                pltpu.sync_copy(mx_hbm.at[pl.ds(base, C)], msg_v)

            @pl.when(c == 1)
            def _():
                pltpu.sync_copy(my_hbm.at[pl.ds(base, C)], msg_v)

            @plsc.parallel_loop(0, C, step=16, unroll=8)
            def _(o):
                plsc.addupdate_scatter(
                    acc_v, [dst_v[pl.ds(o, 16)]], msg_v[pl.ds(o, 16)])

        pltpu.sync_copy(acc_v, part_hbm.at[pl.ds(wid * NPAD, NPAD)])
        if last:
            plsc.subcore_barrier()
            pltpu.sync_copy(
                part_hbm.at[pl.ds(c * 16 * NPAD + sid * SL, SL)], red_v)

            @pl.loop(1, 16)
            def _(p):
                pltpu.sync_copy(
                    part_hbm.at[pl.ds((c * 16 + p) * NPAD + sid * SL, SL)],
                    tmp_v)

                @plsc.parallel_loop(0, SL, step=16, unroll=8)
                def _(o):
                    red_v[pl.ds(o, 16)] = (red_v[pl.ds(o, 16)]
                                           + tmp_v[pl.ds(o, 16)])

            pltpu.sync_copy(red_v, out_hbm.at[pl.ds(c * NPAD + sid * SL, SL)])

    return scatter_kernel, NPAD


def kernel(pos, edge_index, a, W1, b1, W2, b2, W3, b3):
    N = pos.shape[0]
    E = edge_index.shape[1]
    EC = E // P

    px = pos[:, 0]
    py = pos[:, 1]
    a0 = a[:, 0]
    a1 = a[:, 1]
    src = edge_index[0]
    dst = edge_index[1]

    w1bd = jnp.concatenate(
        [W1.T, b1.reshape(-1, 1)],
        axis=1).astype(jnp.bfloat16)
    w2bd = W2.T.astype(jnp.bfloat16)
    b2bd = b2.reshape(-1, 1).astype(jnp.bfloat16)
    w3bd = jnp.concatenate(
        [W3.T, jnp.zeros((6, W3.shape[0]), jnp.float32)],
        axis=0).astype(jnp.bfloat16)
    b3bd = jnp.concatenate(
        [b3, jnp.zeros((6,), jnp.float32)]).reshape(-1, 1)

    mlp = _make_mlp(EC)
    SL = ((-(-N // 16)) + 7) // 8 * 8
    NPAD = SL * 16
    zeros_n = jnp.zeros((NPAD,), jnp.float32)

    accflat = None
    prev = zeros_n
    for p in range(P):
        f4 = _make_gather(N, p * EC, EC)(px, py, a0, a1, src, dst)
        mx, my = mlp(*f4, w1bd, w2bd, b2bd, w3bd, b3bd)
        scatter, _ = _make_scatter(N, p * EC, EC, first=(p == 0),
                                   last=(p == P - 1))
        res = scatter(mx, my, dst, prev)
        prev = res[0]
        if p == P - 1:
            accflat = res[1]

    return accflat.reshape(2, NPAD)[:, :N].T * YNORM


# speedup vs baseline: 1.3285x; 1.3285x over previous
import functools

import jax
import jax.numpy as jnp
from jax import lax
from jax.experimental import pallas as pl
from jax.experimental.pallas import tpu as pltpu
from jax.experimental.pallas import tpu_sc as plsc

MAX_R = 0.1
YNORM = 1.0
P = 5

_SC_MESH = plsc.VectorSubcoreMesh(core_axis_name="c", subcore_axis_name="s")
_SC_PARAMS = pltpu.CompilerParams(needs_layout_passes=False)


def _make_gather(N, off, EC):
    C = 8000
    layout = [(0, 10), (10, 10), (20, 5), (25, 5)]
    for _, cnt in layout:
        assert EC % cnt == 0 and (EC // cnt) % C == 0

    ev = jax.ShapeDtypeStruct((EC,), jnp.float32)

    @functools.partial(
        pl.kernel,
        out_type=(ev, ev, ev, ev),
        mesh=_SC_MESH,
        compiler_params=_SC_PARAMS,
        scratch_types=[
            pltpu.VMEM((N,), jnp.float32),
            pltpu.VMEM((C,), jnp.int32),
            pltpu.VMEM((C,), jnp.int32),
            pltpu.VMEM((C,), jnp.float32),
        ],
    )
    def gather_kernel(px, py, a0h, a1h, src_hbm, dst_hbm,
                      odx, ody, oa0, oa1, tab_v, ids_v, idd_v, out_v):
        wid = lax.axis_index("c") * 16 + lax.axis_index("s")
        tabs = [px, py, a0h, a1h]
        outs = [odx, ody, oa0, oa1]

        for k, (start, cnt) in enumerate(layout):
            @pl.when((wid >= start) & (wid < start + cnt))
            def _(k=k, start=start, cnt=cnt):
                sid = wid - start
                shard = EC // cnt
                nch = shard // C
                pltpu.sync_copy(tabs[k], tab_v)

                @pl.loop(0, nch)
                def _(ci):
                    base = sid * shard + ci * C
                    pltpu.sync_copy(dst_hbm.at[pl.ds(off + base, C)], idd_v)
                    if k < 2:
                        pltpu.sync_copy(src_hbm.at[pl.ds(off + base, C)],
                                        ids_v)

                        @plsc.parallel_loop(0, C, step=16, unroll=8)
                        def _(o):
                            vs = plsc.load_gather(tab_v, [ids_v[pl.ds(o, 16)]])
                            vd = plsc.load_gather(tab_v, [idd_v[pl.ds(o, 16)]])
                            out_v[pl.ds(o, 16)] = (vs - vd) * (1.0 / MAX_R)
                    else:
                        @plsc.parallel_loop(0, C, step=16, unroll=8)
                        def _(o):
                            out_v[pl.ds(o, 16)] = plsc.load_gather(
                                tab_v, [idd_v[pl.ds(o, 16)]])
                    pltpu.sync_copy(out_v, outs[k].at[pl.ds(base, C)])

    return gather_kernel


def _make_mlp(EC):
    BLK = 10240
    Q = BLK // 4
    assert EC % BLK == 0

    def mlp_body(dx_ref, dy_ref, a0_ref, a1_ref,
                 w1_ref, w2_ref, b2_ref, w3_ref, b3_ref,
                 ox_ref, oy_ref):
        d0 = dx_ref[...].reshape(1, BLK)
        d1 = dy_ref[...].reshape(1, BLK)
        r = jnp.sqrt(d0 * d0 + d1 * d1)
        fp = jnp.concatenate(
            [d0, d1, r,
             a0_ref[...].reshape(1, BLK), a1_ref[...].reshape(1, BLK),
             jnp.ones((1, BLK), jnp.float32)], axis=0)
        dn = (((1,), (0,)), ((), ()))
        h = lax.dot_general(w1_ref[...], fp.astype(jnp.bfloat16), dn,
                            preferred_element_type=jnp.float32)
        h = jnp.maximum(h.astype(jnp.bfloat16), 0)
        h = lax.dot_general(w2_ref[...], h, dn,
                            preferred_element_type=jnp.float32)
        h = jnp.maximum(h.astype(jnp.bfloat16) + b2_ref[...], 0)
        o = lax.dot_general(w3_ref[...], h, dn,
                            preferred_element_type=jnp.float32) + b3_ref[...]
        ox_ref[...] = o[0, :]
        oy_ref[...] = o[1, :]

    ev = jax.ShapeDtypeStruct((EC,), jnp.float32)
    edge_spec = pl.BlockSpec((BLK,), lambda i: (i,))
    full = lambda s: pl.BlockSpec(s, lambda i: (0, 0))
    return pl.pallas_call(
        mlp_body,
        grid=(EC // BLK,),
        in_specs=[edge_spec, edge_spec, edge_spec, edge_spec,
                  full((64, 6)), full((64, 64)),
                  full((64, 1)), full((8, 64)), full((8, 1))],
        out_specs=(edge_spec, edge_spec),
        out_shape=(ev, ev),
    )


def _make_scatter(N, off, EC, first, last):
    shard = EC // 16
    C = 8000
    nch = shard // C
    assert shard % C == 0 and C % 16 == 0
    SL = -(-N // 16)
    SL = (SL + 7) // 8 * 8
    NPAD = SL * 16

    parts = jax.ShapeDtypeStruct((32 * NPAD,), jnp.float32)
    out_type = (parts, jax.ShapeDtypeStruct((2 * NPAD,), jnp.float32)) \
        if last else (parts,)

    @functools.partial(
        pl.kernel,
        out_type=out_type,
        mesh=_SC_MESH,
        compiler_params=_SC_PARAMS,
        scratch_types=[
            pltpu.VMEM((NPAD,), jnp.float32),
            pltpu.VMEM((C,), jnp.int32),
            pltpu.VMEM((C,), jnp.float32),
            pltpu.VMEM((SL,), jnp.float32),
            pltpu.VMEM((SL,), jnp.float32),
        ],
    )
    def scatter_kernel(mx_hbm, my_hbm, dst_hbm, init_hbm, *refs):
        if last:
            part_hbm, out_hbm, acc_v, dst_v, msg_v, red_v, tmp_v = refs
        else:
            part_hbm, acc_v, dst_v, msg_v, red_v, tmp_v = refs
        c = lax.axis_index("c")
        sid = lax.axis_index("s")
        wid = c * 16 + sid
        if first:
            pltpu.sync_copy(init_hbm, acc_v)
        else:
            pltpu.sync_copy(init_hbm.at[pl.ds(wid * NPAD, NPAD)], acc_v)

        @pl.loop(0, nch)
        def _(ci):
            base = sid * shard + ci * C
            pltpu.sync_copy(dst_hbm.at[pl.ds(off + base, C)], dst_v)

            @pl.when(c == 0)
            def _():
                pltpu.sync_copy(mx_hbm.at[pl.ds(base, C)], msg_v)

            @pl.when(c == 1)
            def _():
                pltpu.sync_copy(my_hbm.at[pl.ds(base, C)], msg_v)

            @plsc.parallel_loop(0, C, step=16, unroll=8)
            def _(o):
                plsc.addupdate_scatter(
                    acc_v, [dst_v[pl.ds(o, 16)]], msg_v[pl.ds(o, 16)])

        pltpu.sync_copy(acc_v, part_hbm.at[pl.ds(wid * NPAD, NPAD)])
        if last:
            plsc.subcore_barrier()
            pltpu.sync_copy(
                part_hbm.at[pl.ds(c * 16 * NPAD + sid * SL, SL)], red_v)

            @pl.loop(1, 16)
            def _(p):
                pltpu.sync_copy(
                    part_hbm.at[pl.ds((c * 16 + p) * NPAD + sid * SL, SL)],
                    tmp_v)

                @plsc.parallel_loop(0, SL, step=16, unroll=8)
                def _(o):
                    red_v[pl.ds(o, 16)] = (red_v[pl.ds(o, 16)]
                                           + tmp_v[pl.ds(o, 16)])

            pltpu.sync_copy(red_v, out_hbm.at[pl.ds(c * NPAD + sid * SL, SL)])

    return scatter_kernel, NPAD


def kernel(pos, edge_index, a, W1, b1, W2, b2, W3, b3):
    N = pos.shape[0]
    E = edge_index.shape[1]
    EC = E // P

    px = pos[:, 0]
    py = pos[:, 1]
    a0 = a[:, 0]
    a1 = a[:, 1]
    src = edge_index[0]
    dst = edge_index[1]

    w1bd = jnp.concatenate(
        [W1.T, b1.reshape(-1, 1)],
        axis=1).astype(jnp.bfloat16)
    w2bd = W2.T.astype(jnp.bfloat16)
    b2bd = b2.reshape(-1, 1).astype(jnp.bfloat16)
    w3bd = jnp.concatenate(
        [W3.T, jnp.zeros((6, W3.shape[0]), jnp.float32)],
        axis=0).astype(jnp.bfloat16)
    b3bd = jnp.concatenate(
        [b3, jnp.zeros((6,), jnp.float32)]).reshape(-1, 1)

    mlp = _make_mlp(EC)
    SL = ((-(-N // 16)) + 7) // 8 * 8
    NPAD = SL * 16
    zeros_n = jnp.zeros((NPAD,), jnp.float32)

    accflat = None
    prev = zeros_n
    for p in range(P):
        f4 = _make_gather(N, p * EC, EC)(px, py, a0, a1, src, dst)
        mx, my = mlp(*f4, w1bd, w2bd, b2bd, w3bd, b3bd)
        scatter, _ = _make_scatter(N, p * EC, EC, first=(p == 0),
                                   last=(p == P - 1))
        res = scatter(mx, my, dst, prev)
        prev = res[0]
        if p == P - 1:
            accflat = res[1]

    return accflat.reshape(2, NPAD)[:, :N].T * YNORM
